# baseline (speedup 1.0000x reference)
import jax
import jax.numpy as jnp
from jax.experimental import pallas as pl
from jax.experimental.pallas import tpu as pltpu

_N_CODES = 8192
_DIM = 32
_BT = 256
_CHUNK = 2048
_NCHUNK = _N_CODES // _CHUNK
_SUB = 256
_NSUB = _CHUNK // _SUB
_HI = 128
_LO = 64
_BLKW = _LO * _DIM


def _vq_body(xb_ref, eb_ref, x2_ref, e2_ref, b1_ref, b2_ref, b3_ref,
             q_ref, pexp_ref, counts_ref):
    i = pl.program_id(0)
    xb = xb_ref[...]
    x2 = x2_ref[...]

    acc_v = None
    acc_i = None
    for c in range(_NCHUNK):
        m_c = None
        i_c = None
        for s in range(_NSUB):
            off = c * _CHUNK + s * _SUB
            eb = eb_ref[pl.ds(off, _SUB), :]
            nsim2 = jax.lax.dot_general(
                xb, eb, (((1,), (1,)), ((), ())),
                preferred_element_type=jnp.float32)
            e2 = e2_ref[:, pl.ds(off, _SUB)]
            dist = (x2 + e2) + nsim2
            ms = jnp.min(dist, axis=1, keepdims=True)
            iota = jax.lax.broadcasted_iota(jnp.int32, dist.shape, 1)
            idxs = jnp.min(jnp.where(dist <= ms, iota + off, _N_CODES),
                           axis=1, keepdims=True)
            if m_c is None:
                m_c, i_c = ms, idxs
            else:
                upd = ms < m_c
                m_c = jnp.where(upd, ms, m_c)
                i_c = jnp.where(upd, idxs, i_c)
        mq = m_c.astype(jnp.bfloat16).astype(jnp.float32)
        if acc_v is None:
            acc_v, acc_i = mq, i_c
        else:
            upd = m_c < acc_v
            acc_v = jnp.where(upd, mq, acc_v)
            acc_i = jnp.where(upd, i_c, acc_i)

    hi = jax.lax.shift_right_logical(acc_i, 6)
    lo = jax.lax.bitwise_and(acc_i, 63)
    iota_hi = jax.lax.broadcasted_iota(jnp.int32, (_BT, _HI), 1)
    iota_lo = jax.lax.broadcasted_iota(jnp.int32, (_BT, _LO), 1)
    oh_hi = jnp.where(iota_hi == hi, 1.0, 0.0)
    oh_lo = jnp.where(iota_lo == lo, 1.0, 0.0)

    dn = (((1,), (0,)), ((), ()))
    inter = jax.lax.dot_general(
        oh_hi, b1_ref[...], dn, preferred_element_type=jnp.float32)
    inter = inter + jax.lax.dot_general(
        oh_hi, b2_ref[...], dn, preferred_element_type=jnp.float32)
    inter = inter + jax.lax.dot_general(
        oh_hi, b3_ref[...], dn, preferred_element_type=jnp.float32)
    iota_w = jax.lax.broadcasted_iota(jnp.int32, (_BT, _BLKW), 1)
    seg = jax.lax.shift_right_logical(iota_w, 5)
    masked = jnp.where(seg == lo, inter, 0.0)
    w = _BLKW
    while w > _DIM:
        w //= 2
        masked = masked[:, :w] + masked[:, w:2 * w]
    q_ref[...] = masked

    @pl.when(i == 0)
    def _zero():
        counts_ref[...] = jnp.zeros_like(counts_ref)

    counts_ref[...] += jax.lax.dot_general(
        oh_hi, oh_lo, (((0,), (0,)), ((), ())),
        preferred_element_type=jnp.float32)

    @pl.when(i == pl.num_programs(0) - 1)
    def _finish():
        n_tok = pl.num_programs(0) * _BT
        p = counts_ref[...] / n_tok
        ent = -jnp.sum(p * jnp.log(p + 1e-10))
        pexp_ref[...] = jnp.exp(ent).reshape(1, 1)


def kernel(x, embeddings):
    input_shape = x.shape
    flat = x.reshape(-1, _DIM)
    n_tok = flat.shape[0]
    x2 = jnp.sum(flat ** 2, axis=1, keepdims=True)
    e2 = jnp.sum(embeddings ** 2, axis=1)[None, :]
    xb = flat.astype(jnp.bfloat16) * jnp.bfloat16(-2.0)
    ebf = embeddings.astype(jnp.bfloat16)
    e_blocks = embeddings.reshape(_HI, _BLKW)
    b1 = e_blocks.astype(jnp.bfloat16).astype(jnp.float32)
    r1 = e_blocks - b1
    b2 = r1.astype(jnp.bfloat16).astype(jnp.float32)
    b3 = (r1 - b2).astype(jnp.bfloat16).astype(jnp.float32)
    grid = (n_tok // _BT,)
    q, pexp = pl.pallas_call(
        _vq_body,
        grid=grid,
        in_specs=[
            pl.BlockSpec((_BT, _DIM), lambda i: (i, 0)),
            pl.BlockSpec((_N_CODES, _DIM), lambda i: (0, 0)),
            pl.BlockSpec((_BT, 1), lambda i: (i, 0)),
            pl.BlockSpec((1, _N_CODES), lambda i: (0, 0)),
            pl.BlockSpec((_HI, _BLKW), lambda i: (0, 0)),
            pl.BlockSpec((_HI, _BLKW), lambda i: (0, 0)),
            pl.BlockSpec((_HI, _BLKW), lambda i: (0, 0)),
        ],
        out_specs=[
            pl.BlockSpec((_BT, _DIM), lambda i: (i, 0)),
            pl.BlockSpec((1, 1), lambda i: (0, 0)),
        ],
        out_shape=[
            jax.ShapeDtypeStruct((n_tok, _DIM), jnp.float32),
            jax.ShapeDtypeStruct((1, 1), jnp.float32),
        ],
        scratch_shapes=[pltpu.VMEM((_HI, _LO), jnp.float32)],
    )(xb, ebf, x2, e2, b1, b2, b3)
    return q.reshape(input_shape), pexp[0, 0]


# speedup vs baseline: 1.4222x; 1.4222x over previous
import jax
import jax.numpy as jnp
from jax.experimental import pallas as pl
from jax.experimental.pallas import tpu as pltpu

_N_CODES = 8192
_DIM = 32
_BT = 256
_CHUNK = 2048
_NCHUNK = _N_CODES // _CHUNK
_SUB = 1024
_NSUB = _CHUNK // _SUB
_HI = 128
_LO = 64
_BLKW = _LO * _DIM


def _vq_body(xb_ref, eb_ref, x2_ref, e2_ref, b1_ref,
             q_ref, pexp_ref, counts_ref):
    i = pl.program_id(0)
    xb = xb_ref[...]
    x2 = x2_ref[...]

    acc_v = None
    acc_i = None
    for c in range(_NCHUNK):
        m_c = None
        i_c = None
        for s in range(_NSUB):
            off = c * _CHUNK + s * _SUB
            eb = eb_ref[pl.ds(off, _SUB), :]
            nsim2 = jax.lax.dot_general(
                xb, eb, (((1,), (1,)), ((), ())),
                preferred_element_type=jnp.float32)
            e2 = e2_ref[:, pl.ds(off, _SUB)]
            dist = (x2 + e2) + nsim2
            ms = jnp.min(dist, axis=1, keepdims=True)
            iota = jax.lax.broadcasted_iota(jnp.int32, dist.shape, 1)
            idxs = jnp.min(jnp.where(dist <= ms, iota + off, _N_CODES),
                           axis=1, keepdims=True)
            if m_c is None:
                m_c, i_c = ms, idxs
            else:
                upd = ms < m_c
                m_c = jnp.where(upd, ms, m_c)
                i_c = jnp.where(upd, idxs, i_c)
        mq = m_c.astype(jnp.bfloat16).astype(jnp.float32)
        if acc_v is None:
            acc_v, acc_i = mq, i_c
        else:
            upd = m_c < acc_v
            acc_v = jnp.where(upd, mq, acc_v)
            acc_i = jnp.where(upd, i_c, acc_i)

    hi = jax.lax.shift_right_logical(acc_i, 6)
    lo = jax.lax.bitwise_and(acc_i, 63)
    iota_hi = jax.lax.broadcasted_iota(jnp.int32, (_BT, _HI), 1)
    iota_lo = jax.lax.broadcasted_iota(jnp.int32, (_BT, _LO), 1)
    oh_hi = jnp.where(iota_hi == hi, 1.0, 0.0)
    oh_lo = jnp.where(iota_lo == lo, 1.0, 0.0)

    dn = (((1,), (0,)), ((), ()))
    inter = jax.lax.dot_general(
        oh_hi, b1_ref[...], dn,
        precision=jax.lax.Precision.HIGHEST,
        preferred_element_type=jnp.float32)
    iota_w = jax.lax.broadcasted_iota(jnp.int32, (_BT, _BLKW), 1)
    seg = jax.lax.shift_right_logical(iota_w, 5)
    masked = jnp.where(seg == lo, inter, 0.0)
    w = _BLKW
    while w > _DIM:
        w //= 2
        masked = masked[:, :w] + masked[:, w:2 * w]
    q_ref[...] = masked

    @pl.when(i == 0)
    def _zero():
        counts_ref[...] = jnp.zeros_like(counts_ref)

    counts_ref[...] += jax.lax.dot_general(
        oh_hi, oh_lo, (((0,), (0,)), ((), ())),
        preferred_element_type=jnp.float32)

    @pl.when(i == pl.num_programs(0) - 1)
    def _finish():
        n_tok = pl.num_programs(0) * _BT
        p = counts_ref[...] / n_tok
        ent = -jnp.sum(p * jnp.log(p + 1e-10))
        pexp_ref[...] = jnp.exp(ent).reshape(1, 1)


def kernel(x, embeddings):
    input_shape = x.shape
    flat = x.reshape(-1, _DIM)
    n_tok = flat.shape[0]
    x2 = jnp.sum(flat ** 2, axis=1, keepdims=True)
    e2 = jnp.sum(embeddings ** 2, axis=1)[None, :]
    xb = flat.astype(jnp.bfloat16) * jnp.bfloat16(-2.0)
    ebf = embeddings.astype(jnp.bfloat16)
    b1 = embeddings.reshape(_HI, _BLKW)
    grid = (n_tok // _BT,)
    q, pexp = pl.pallas_call(
        _vq_body,
        grid=grid,
        in_specs=[
            pl.BlockSpec((_BT, _DIM), lambda i: (i, 0)),
            pl.BlockSpec((_N_CODES, _DIM), lambda i: (0, 0)),
            pl.BlockSpec((_BT, 1), lambda i: (i, 0)),
            pl.BlockSpec((1, _N_CODES), lambda i: (0, 0)),
            pl.BlockSpec((_HI, _BLKW), lambda i: (0, 0)),
        ],
        out_specs=[
            pl.BlockSpec((_BT, _DIM), lambda i: (i, 0)),
            pl.BlockSpec((1, 1), lambda i: (0, 0)),
        ],
        out_shape=[
            jax.ShapeDtypeStruct((n_tok, _DIM), jnp.float32),
            jax.ShapeDtypeStruct((1, 1), jnp.float32),
        ],
        scratch_shapes=[pltpu.VMEM((_HI, _LO), jnp.float32)],
    )(xb, ebf, x2, e2, b1)
    return q.reshape(input_shape), pexp[0, 0]


# speedup vs baseline: 1.5703x; 1.1041x over previous
import jax
import jax.numpy as jnp
from jax.experimental import pallas as pl
from jax.experimental.pallas import tpu as pltpu

_N_CODES = 8192
_DIM = 32
_BT = 512
_CHUNK = 2048
_NCHUNK = _N_CODES // _CHUNK
_SUB = 2048
_NSUB = _CHUNK // _SUB
_HI = 128
_LO = 64
_BLKW = _LO * _DIM


def _vq_body(xb_ref, eb_ref, x2_ref, e2_ref, b1_ref,
             q_ref, pexp_ref, counts_ref):
    i = pl.program_id(0)
    xb = xb_ref[...]
    x2 = x2_ref[...]

    acc_v = None
    acc_i = None
    for c in range(_NCHUNK):
        m_c = None
        i_c = None
        for s in range(_NSUB):
            off = c * _CHUNK + s * _SUB
            eb = eb_ref[pl.ds(off, _SUB), :]
            nsim2 = jax.lax.dot_general(
                xb, eb, (((1,), (1,)), ((), ())),
                preferred_element_type=jnp.float32)
            e2 = e2_ref[:, pl.ds(off, _SUB)]
            dist = (x2 + e2) + nsim2
            ms = jnp.min(dist, axis=1, keepdims=True)
            iota = jax.lax.broadcasted_iota(jnp.int32, dist.shape, 1)
            idxs = jnp.min(jnp.where(dist <= ms, iota + off, _N_CODES),
                           axis=1, keepdims=True)
            if m_c is None:
                m_c, i_c = ms, idxs
            else:
                upd = ms < m_c
                m_c = jnp.where(upd, ms, m_c)
                i_c = jnp.where(upd, idxs, i_c)
        mq = m_c.astype(jnp.bfloat16).astype(jnp.float32)
        if acc_v is None:
            acc_v, acc_i = mq, i_c
        else:
            upd = m_c < acc_v
            acc_v = jnp.where(upd, mq, acc_v)
            acc_i = jnp.where(upd, i_c, acc_i)

    hi = jax.lax.shift_right_logical(acc_i, 6)
    lo = jax.lax.bitwise_and(acc_i, 63)
    iota_hi = jax.lax.broadcasted_iota(jnp.int32, (_BT, _HI), 1)
    iota_lo = jax.lax.broadcasted_iota(jnp.int32, (_BT, _LO), 1)
    oh_hi = jnp.where(iota_hi == hi, 1.0, 0.0)
    oh_lo = jnp.where(iota_lo == lo, 1.0, 0.0)

    dn = (((1,), (0,)), ((), ()))
    inter = jax.lax.dot_general(
        oh_hi, b1_ref[...], dn,
        precision=jax.lax.Precision.HIGHEST,
        preferred_element_type=jnp.float32)
    iota_w = jax.lax.broadcasted_iota(jnp.int32, (_BT, _BLKW), 1)
    seg = jax.lax.shift_right_logical(iota_w, 5)
    masked = jnp.where(seg == lo, inter, 0.0)
    w = _BLKW
    while w > _DIM:
        w //= 2
        masked = masked[:, :w] + masked[:, w:2 * w]
    q_ref[...] = masked

    @pl.when(i == 0)
    def _zero():
        counts_ref[...] = jnp.zeros_like(counts_ref)

    counts_ref[...] += jax.lax.dot_general(
        oh_hi, oh_lo, (((0,), (0,)), ((), ())),
        preferred_element_type=jnp.float32)

    @pl.when(i == pl.num_programs(0) - 1)
    def _finish():
        n_tok = pl.num_programs(0) * _BT
        p = counts_ref[...] / n_tok
        ent = -jnp.sum(p * jnp.log(p + 1e-10))
        pexp_ref[...] = jnp.exp(ent).reshape(1, 1)


def kernel(x, embeddings):
    input_shape = x.shape
    flat = x.reshape(-1, _DIM)
    n_tok = flat.shape[0]
    x2 = jnp.sum(flat ** 2, axis=1, keepdims=True)
    e2 = jnp.sum(embeddings ** 2, axis=1)[None, :]
    xb = flat.astype(jnp.bfloat16) * jnp.bfloat16(-2.0)
    ebf = embeddings.astype(jnp.bfloat16)
    b1 = embeddings.reshape(_HI, _BLKW)
    grid = (n_tok // _BT,)
    q, pexp = pl.pallas_call(
        _vq_body,
        grid=grid,
        in_specs=[
            pl.BlockSpec((_BT, _DIM), lambda i: (i, 0)),
            pl.BlockSpec((_N_CODES, _DIM), lambda i: (0, 0)),
            pl.BlockSpec((_BT, 1), lambda i: (i, 0)),
            pl.BlockSpec((1, _N_CODES), lambda i: (0, 0)),
            pl.BlockSpec((_HI, _BLKW), lambda i: (0, 0)),
        ],
        out_specs=[
            pl.BlockSpec((_BT, _DIM), lambda i: (i, 0)),
            pl.BlockSpec((1, 1), lambda i: (0, 0)),
        ],
        out_shape=[
            jax.ShapeDtypeStruct((n_tok, _DIM), jnp.float32),
            jax.ShapeDtypeStruct((1, 1), jnp.float32),
        ],
        scratch_shapes=[pltpu.VMEM((_HI, _LO), jnp.float32)],
    )(xb, ebf, x2, e2, b1)
    return q.reshape(input_shape), pexp[0, 0]


# speedup vs baseline: 1.5835x; 1.0084x over previous
import jax
import jax.numpy as jnp
from jax.experimental import pallas as pl
from jax.experimental.pallas import tpu as pltpu

_N_CODES = 8192
_DIM = 32
_BT = 1024
_CHUNK = 2048
_NCHUNK = _N_CODES // _CHUNK
_SUB = 2048
_NSUB = _CHUNK // _SUB
_HI = 128
_LO = 64
_BLKW = _LO * _DIM


def _vq_body(xb_ref, eb_ref, x2_ref, e2_ref, b1_ref,
             q_ref, pexp_ref, counts_ref):
    i = pl.program_id(0)
    xb = xb_ref[...]
    x2 = x2_ref[...]

    acc_v = None
    acc_i = None
    for c in range(_NCHUNK):
        m_c = None
        i_c = None
        for s in range(_NSUB):
            off = c * _CHUNK + s * _SUB
            eb = eb_ref[pl.ds(off, _SUB), :]
            nsim2 = jax.lax.dot_general(
                xb, eb, (((1,), (1,)), ((), ())),
                preferred_element_type=jnp.float32)
            e2 = e2_ref[:, pl.ds(off, _SUB)]
            dist = (x2 + e2) + nsim2
            ms = jnp.min(dist, axis=1, keepdims=True)
            iota = jax.lax.broadcasted_iota(jnp.int32, dist.shape, 1)
            idxs = jnp.min(jnp.where(dist <= ms, iota + off, _N_CODES),
                           axis=1, keepdims=True)
            if m_c is None:
                m_c, i_c = ms, idxs
            else:
                upd = ms < m_c
                m_c = jnp.where(upd, ms, m_c)
                i_c = jnp.where(upd, idxs, i_c)
        mq = m_c.astype(jnp.bfloat16).astype(jnp.float32)
        if acc_v is None:
            acc_v, acc_i = mq, i_c
        else:
            upd = m_c < acc_v
            acc_v = jnp.where(upd, mq, acc_v)
            acc_i = jnp.where(upd, i_c, acc_i)

    hi = jax.lax.shift_right_logical(acc_i, 6)
    lo = jax.lax.bitwise_and(acc_i, 63)
    iota_hi = jax.lax.broadcasted_iota(jnp.int32, (_BT, _HI), 1)
    iota_lo = jax.lax.broadcasted_iota(jnp.int32, (_BT, _LO), 1)
    oh_hi = jnp.where(iota_hi == hi, 1.0, 0.0)
    oh_lo = jnp.where(iota_lo == lo, 1.0, 0.0)

    dn = (((1,), (0,)), ((), ()))
    inter = jax.lax.dot_general(
        oh_hi, b1_ref[...], dn,
        precision=jax.lax.Precision.HIGHEST,
        preferred_element_type=jnp.float32)
    iota_w = jax.lax.broadcasted_iota(jnp.int32, (_BT, _BLKW), 1)
    seg = jax.lax.shift_right_logical(iota_w, 5)
    masked = jnp.where(seg == lo, inter, 0.0)
    w = _BLKW
    while w > _DIM:
        w //= 2
        masked = masked[:, :w] + masked[:, w:2 * w]
    q_ref[...] = masked

    @pl.when(i == 0)
    def _zero():
        counts_ref[...] = jnp.zeros_like(counts_ref)

    counts_ref[...] += jax.lax.dot_general(
        oh_hi, oh_lo, (((0,), (0,)), ((), ())),
        preferred_element_type=jnp.float32)

    @pl.when(i == pl.num_programs(0) - 1)
    def _finish():
        n_tok = pl.num_programs(0) * _BT
        p = counts_ref[...] / n_tok
        ent = -jnp.sum(p * jnp.log(p + 1e-10))
        pexp_ref[...] = jnp.exp(ent).reshape(1, 1)


def kernel(x, embeddings):
    input_shape = x.shape
    flat = x.reshape(-1, _DIM)
    n_tok = flat.shape[0]
    x2 = jnp.sum(flat ** 2, axis=1, keepdims=True)
    e2 = jnp.sum(embeddings ** 2, axis=1)[None, :]
    xb = flat.astype(jnp.bfloat16) * jnp.bfloat16(-2.0)
    ebf = embeddings.astype(jnp.bfloat16)
    b1 = embeddings.reshape(_HI, _BLKW)
    grid = (n_tok // _BT,)
    q, pexp = pl.pallas_call(
        _vq_body,
        grid=grid,
        in_specs=[
            pl.BlockSpec((_BT, _DIM), lambda i: (i, 0)),
            pl.BlockSpec((_N_CODES, _DIM), lambda i: (0, 0)),
            pl.BlockSpec((_BT, 1), lambda i: (i, 0)),
            pl.BlockSpec((1, _N_CODES), lambda i: (0, 0)),
            pl.BlockSpec((_HI, _BLKW), lambda i: (0, 0)),
        ],
        out_specs=[
            pl.BlockSpec((_BT, _DIM), lambda i: (i, 0)),
            pl.BlockSpec((1, 1), lambda i: (0, 0)),
        ],
        out_shape=[
            jax.ShapeDtypeStruct((n_tok, _DIM), jnp.float32),
            jax.ShapeDtypeStruct((1, 1), jnp.float32),
        ],
        scratch_shapes=[pltpu.VMEM((_HI, _LO), jnp.float32)],
    )(xb, ebf, x2, e2, b1)
    return q.reshape(input_shape), pexp[0, 0]
